# Initial kernel scaffold; baseline (speedup 1.0000x reference)
import jax
import jax.numpy as jnp
from jax.experimental import pallas as pl


def kernel(x, edge_index, h_prev, W, gamma, beta, Wz, Uz, bz, Wr, Ur, br, Wh, Uh, bh):
    raise NotImplementedError("write your pallas kernel here")


# speedup vs baseline: 4.7850x; 4.7850x over previous
import functools

import jax
import jax.numpy as jnp
from jax import lax
from jax.experimental import pallas as pl
from jax.experimental.pallas import tpu as pltpu
from jax.experimental.pallas import tpu_sc as plsc

NC = 2
NS = 16
C = 80
L = 16


def _sc_aggregate(x, src, dst, z_agg):
    n, d = x.shape
    e = src.shape[0]
    ept = e // (NC * NS)
    rpt = (n // NS) // 8 * 8
    rlast = n - (NS - 1) * rpt

    mesh = plsc.VectorSubcoreMesh(core_axis_name="c", subcore_axis_name="s")

    @functools.partial(
        pl.kernel,
        mesh=mesh,
        out_type=[
            jax.ShapeDtypeStruct((NC, n, d), jnp.float32),
            jax.ShapeDtypeStruct((NC, n, d), jnp.float32),
        ],
        scratch_types=[
            pltpu.VMEM((C,), jnp.int32),
            pltpu.VMEM((C,), jnp.int32),
            pltpu.VMEM((C, d), jnp.float32),
            pltpu.VMEM((C, d), jnp.float32),
            pltpu.VMEM_SHARED((n, d), jnp.float32),
            pltpu.SemaphoreType.DMA,
        ],
    )
    def sc_kern(x_hbm, src_hbm, dst_hbm, zagg_hbm,
                agg_out, deg_out, idx_s, idx_d, rows_v, ones_v, agg_sh, sem):
        c = lax.axis_index("c")
        s = lax.axis_index("s")
        r0 = s * rpt
        base = (c * NS + s) * ept

        def zero_acc():
            @pl.when(s < NS - 1)
            def _():
                pltpu.sync_copy(zagg_hbm.at[pl.ds(r0, rpt)],
                                agg_sh.at[pl.ds(r0, rpt)])

            @pl.when(s == NS - 1)
            def _():
                pltpu.sync_copy(zagg_hbm.at[pl.ds(r0, rlast)],
                                agg_sh.at[pl.ds(r0, rlast)])

        def writeback(out_ref):
            @pl.when(s < NS - 1)
            def _():
                pltpu.sync_copy(agg_sh.at[pl.ds(r0, rpt)],
                                out_ref.at[c, pl.ds(r0, rpt)])

            @pl.when(s == NS - 1)
            def _():
                pltpu.sync_copy(agg_sh.at[pl.ds(r0, rlast)],
                                out_ref.at[c, pl.ds(r0, rlast)])

        zero_acc()

        def fill_row(i, carry):
            def fill_lane(j, carry2):
                ones_v[i, pl.ds(j * L, L)] = jnp.full((L,), 1.0, jnp.float32)
                return carry2
            return lax.fori_loop(0, d // L, fill_lane, carry)

        lax.fori_loop(0, C, fill_row, 0)
        plsc.subcore_barrier()

        def body1(g, carry):
            off = base + g * C
            pltpu.sync_copy(src_hbm.at[pl.ds(off, C)], idx_s)
            pltpu.sync_copy(dst_hbm.at[pl.ds(off, C)], idx_d)
            pltpu.async_copy(x_hbm.at[idx_s], rows_v, sem).wait()
            pltpu.sync_copy(rows_v, agg_sh.at[idx_d], add=True)
            return carry

        lax.fori_loop(0, ept // C, body1, 0)
        plsc.subcore_barrier()

        writeback(agg_out)
        zero_acc()
        plsc.subcore_barrier()

        def body2(g, carry):
            off = base + g * C
            pltpu.sync_copy(dst_hbm.at[pl.ds(off, C)], idx_d)
            pltpu.sync_copy(ones_v, agg_sh.at[idx_d], add=True)
            return carry

        lax.fori_loop(0, ept // C, body2, 0)
        plsc.subcore_barrier()

        writeback(deg_out)

    return sc_kern(x, src, dst, z_agg)


def _tc_linear_stats(agg_p, deg_p, W, block):
    _, n, d = agg_p.shape
    nb = n // block

    def body(p_ref, d_ref, w_ref, h_ref, s_ref, q_ref):
        i = pl.program_id(0)
        agg = p_ref[0] + p_ref[1]
        deg = d_ref[0, :, 0:1] + d_ref[1, :, 0:1]
        deg = jnp.maximum(deg, 1.0)
        h = jnp.dot(agg / deg, w_ref[...], preferred_element_type=jnp.float32)
        h_ref[...] = h
        s8 = jnp.sum(h.reshape(block // 8, 8, d), axis=0)
        q8 = jnp.sum((h * h).reshape(block // 8, 8, d), axis=0)

        @pl.when(i == 0)
        def _():
            s_ref[...] = s8
            q_ref[...] = q8

        @pl.when(i != 0)
        def _():
            s_ref[...] += s8
            q_ref[...] += q8

    return pl.pallas_call(
        body,
        grid=(nb,),
        in_specs=[
            pl.BlockSpec((NC, block, d), lambda i: (0, i, 0)),
            pl.BlockSpec((NC, block, d), lambda i: (0, i, 0)),
            pl.BlockSpec((d, d), lambda i: (0, 0)),
        ],
        out_specs=[
            pl.BlockSpec((block, d), lambda i: (i, 0)),
            pl.BlockSpec((8, d), lambda i: (0, 0)),
            pl.BlockSpec((8, d), lambda i: (0, 0)),
        ],
        out_shape=[
            jax.ShapeDtypeStruct((n, d), jnp.float32),
            jax.ShapeDtypeStruct((8, d), jnp.float32),
            jax.ShapeDtypeStruct((8, d), jnp.float32),
        ],
    )(agg_p, deg_p, W)


def _tc_bn_gru(h_pre, h_prev, sums, sumsq, gamma, beta,
               Wz, Uz, bz, Wr, Ur, br, Wh, Uh, bh, block):
    n, d = h_pre.shape
    nb = n // block
    inv_n = 1.0 / n

    def body(h_ref, hp_ref, s_ref, q_ref, g_ref, b_ref,
             wz_ref, uz_ref, bz_ref, wr_ref, ur_ref, br_ref,
             wh_ref, uh_ref, bh_ref, o_ref):
        mu = jnp.sum(s_ref[...], axis=0, keepdims=True) * inv_n
        ex2 = jnp.sum(q_ref[...], axis=0, keepdims=True) * inv_n
        var = ex2 - mu * mu
        inv = lax.rsqrt(var + 1e-5)
        h = (h_ref[...] - mu) * (inv * g_ref[...]) + b_ref[...]
        h = jnp.maximum(h, 0.0)
        hp = hp_ref[...]
        dot = lambda a, b: jnp.dot(a, b, preferred_element_type=jnp.float32)
        z = jax.nn.sigmoid(dot(h, wz_ref[...]) + dot(hp, uz_ref[...])
                           + bz_ref[...])
        r = jax.nn.sigmoid(dot(h, wr_ref[...]) + dot(hp, ur_ref[...])
                           + br_ref[...])
        ht = jnp.tanh(dot(h, wh_ref[...]) + dot(r * hp, uh_ref[...])
                      + bh_ref[...])
        o_ref[...] = (1.0 - z) * hp + z * ht

    blk = pl.BlockSpec((block, d), lambda i: (i, 0))
    small = pl.BlockSpec((8, d), lambda i: (0, 0))
    row = pl.BlockSpec((1, d), lambda i: (0, 0))
    mat = pl.BlockSpec((d, d), lambda i: (0, 0))
    return pl.pallas_call(
        body,
        grid=(nb,),
        in_specs=[blk, blk, small, small, row, row,
                  mat, mat, row, mat, mat, row, mat, mat, row],
        out_specs=blk,
        out_shape=jax.ShapeDtypeStruct((n, d), jnp.float32),
    )(h_pre, h_prev, sums, sumsq, gamma.reshape(1, d), beta.reshape(1, d),
      Wz, Uz, bz.reshape(1, d), Wr, Ur, br.reshape(1, d),
      Wh, Uh, bh.reshape(1, d))


def kernel(x, edge_index, h_prev, W, gamma, beta,
           Wz, Uz, bz, Wr, Ur, br, Wh, Uh, bh):
    n, d = x.shape
    e = edge_index.shape[1]
    assert e % (NC * NS * C) == 0 and n % 8 == 0

    src = edge_index[0].astype(jnp.int32)
    dst = edge_index[1].astype(jnp.int32)
    z_agg = jnp.zeros((n, d), jnp.float32)

    agg_p, deg_p = _sc_aggregate(x, src, dst, z_agg)

    block = 1000
    h_pre, sums, sumsq = _tc_linear_stats(agg_p, deg_p, W, block)
    return _tc_bn_gru(h_pre, h_prev, sums, sumsq, gamma, beta,
                      Wz, Uz, bz, Wr, Ur, br, Wh, Uh, bh, block)
